# Initial kernel scaffold; baseline (speedup 1.0000x reference)
import jax
import jax.numpy as jnp
from jax.experimental import pallas as pl


def kernel(x):
    raise NotImplementedError("write your pallas kernel here")


# speedup vs baseline: 1.1588x; 1.1588x over previous
import jax
import jax.numpy as jnp
from jax.experimental import pallas as pl

DEPTH = 1000
ROWS_PER_BLOCK = 256


def _body(x_ref, o_ref):
    xm = x_ref[0, 0, :] % DEPTH
    iota = jax.lax.broadcasted_iota(jnp.int32, (ROWS_PER_BLOCK, DEPTH), 1)
    o_ref[0] = (iota == xm[:, None]).astype(jnp.float32)


def kernel(x):
    B, T = x.shape
    n = B * T
    R = ROWS_PER_BLOCK
    nblk = n // R
    xf = x.reshape(nblk, 1, R)
    out = pl.pallas_call(
        _body,
        grid=(nblk,),
        in_specs=[pl.BlockSpec((1, 1, R), lambda i: (i, 0, 0))],
        out_specs=pl.BlockSpec((1, R, DEPTH), lambda i: (i, 0, 0)),
        out_shape=jax.ShapeDtypeStruct((nblk, R, DEPTH), jnp.float32),
    )(xf)
    return out.reshape(B, T, DEPTH)
